# Initial kernel scaffold; baseline (speedup 1.0000x reference)
import jax
import jax.numpy as jnp
from jax.experimental import pallas as pl


def kernel(input, route_index, weight, weight_fact):
    raise NotImplementedError("write your pallas kernel here")


# speedup vs baseline: 1.3506x; 1.3506x over previous
import functools

import jax
import jax.numpy as jnp
from jax import lax
from jax.experimental import pallas as pl
from jax.experimental.pallas import tpu as pltpu
from jax.experimental.pallas import tpu_sc as plsc

B = 16384
C = 128
LANES = 16
NCORES = 2
NSUBCORES = 16
NUM_WORKERS = NCORES * NSUBCORES
ROWS_PER_WORKER = B // NUM_WORKERS
CHUNK = 128
NCHUNKS = ROWS_PER_WORKER // CHUNK
CVECS = C // LANES


def _sc_body(in_hbm, idx_hbm, w_hbm, fact_hbm, out_hbm,
             idx_v, in_v, sl_v, fact_v, sem_in, sem_sl):
    wid = lax.axis_index("s") * NCORES + lax.axis_index("c")
    row0 = wid * ROWS_PER_WORKER

    pltpu.sync_copy(idx_hbm.at[pl.ds(wid * NCHUNKS, NCHUNKS), :], idx_v)
    pltpu.sync_copy(fact_hbm, fact_v)
    fact_vs = [fact_v[0, pl.ds(j * LANES, LANES)] for j in range(CVECS)]

    for g in range(NCHUNKS):
        r0 = row0 + g * CHUNK
        cp_in = pltpu.async_copy(in_hbm.at[pl.ds(r0, CHUNK), :], in_v, sem_in)
        cp_sl = pltpu.async_copy(w_hbm.at[idx_v.at[g]], sl_v, sem_sl)
        cp_in.wait()
        cp_sl.wait()

        def row_body(r, carry):
            for j in range(CVECS):
                sl = pl.ds(j * LANES, LANES)
                iv = in_v[r, sl]
                sv = sl_v[r, sl]
                in_v[r, sl] = jnp.where(iv >= 0.0, iv, (sv + fact_vs[j]) * iv)
            return carry

        lax.fori_loop(0, CHUNK, row_body, 0, unroll=False)

        pltpu.sync_copy(in_v, out_hbm.at[pl.ds(r0, CHUNK), :])


@functools.partial(jax.jit, static_argnames=())
def _run(input, route_index, weight, weight_fact):
    mesh = plsc.VectorSubcoreMesh(core_axis_name="c", subcore_axis_name="s")
    f = functools.partial(
        pl.kernel,
        out_type=jax.ShapeDtypeStruct((B, C), jnp.float32),
        mesh=mesh,
        scratch_types=[
            pltpu.VMEM((NCHUNKS, CHUNK), jnp.int32),
            pltpu.VMEM((CHUNK, C), jnp.float32),
            pltpu.VMEM((CHUNK, C), jnp.float32),
            pltpu.VMEM((1, C), jnp.float32),
            pltpu.SemaphoreType.DMA,
            pltpu.SemaphoreType.DMA,
        ],
    )(_sc_body)
    idx2d = route_index.astype(jnp.int32).reshape(NUM_WORKERS * NCHUNKS, CHUNK)
    return f(input, idx2d, weight, weight_fact)


def kernel(input, route_index, weight, weight_fact):
    return _run(input, route_index, weight, weight_fact)


# speedup vs baseline: 1.3620x; 1.0085x over previous
import functools

import jax
import jax.numpy as jnp
from jax import lax
from jax.experimental import pallas as pl
from jax.experimental.pallas import tpu as pltpu
from jax.experimental.pallas import tpu_sc as plsc

B = 16384
C = 128
LANES = 16
NCORES = 2
NSUBCORES = 16
NUM_WORKERS = NCORES * NSUBCORES
ROWS_PER_WORKER = B // NUM_WORKERS
CHUNK = 128
NCHUNKS = ROWS_PER_WORKER // CHUNK
CVECS = C // LANES


def _sc_body(in_hbm, idx_hbm, w_hbm, fact_hbm, out_hbm,
             idx_v, in_v, sl_v, fact_v,
             sem_in0, sem_in1, sem_sl0, sem_sl1, sem_out0, sem_out1):
    wid = lax.axis_index("s") * NCORES + lax.axis_index("c")
    row0 = wid * ROWS_PER_WORKER
    sems_in = (sem_in0, sem_in1)
    sems_sl = (sem_sl0, sem_sl1)
    sems_out = (sem_out0, sem_out1)

    pltpu.sync_copy(idx_hbm.at[pl.ds(wid * NCHUNKS, NCHUNKS), :], idx_v)
    pltpu.sync_copy(fact_hbm, fact_v)
    fact_vs = [fact_v[0, pl.ds(j * LANES, LANES)] for j in range(CVECS)]

    def start(g):
        s = g % 2
        r0 = row0 + g * CHUNK
        cin = pltpu.async_copy(in_hbm.at[pl.ds(r0, CHUNK), :], in_v.at[s],
                               sems_in[s])
        csl = pltpu.async_copy(w_hbm.at[idx_v.at[g]], sl_v.at[s], sems_sl[s])
        return cin, csl

    def compute(s):
        def row_body(r, carry):
            for j in range(CVECS):
                sl = pl.ds(j * LANES, LANES)
                iv = in_v[s, r, sl]
                sv = sl_v[s, r, sl]
                in_v[s, r, sl] = jnp.where(iv >= 0.0, iv,
                                           (sv + fact_vs[j]) * iv)
            return carry

        lax.fori_loop(0, CHUNK, row_body, 0, unroll=False)

    cps = {}
    outs = {}
    cps[0] = start(0)
    for g in range(NCHUNKS):
        if g + 1 < NCHUNKS:
            if g >= 1:
                outs[g - 1].wait()
            cps[g + 1] = start(g + 1)
        cin, csl = cps.pop(g)
        cin.wait()
        csl.wait()
        s = g % 2
        compute(s)
        outs[g] = pltpu.async_copy(in_v.at[s],
                                   out_hbm.at[pl.ds(row0 + g * CHUNK, CHUNK), :],
                                   sems_out[s])
    outs[NCHUNKS - 2].wait()
    outs[NCHUNKS - 1].wait()


@functools.partial(jax.jit, static_argnames=())
def _run(input, route_index, weight, weight_fact):
    mesh = plsc.VectorSubcoreMesh(core_axis_name="c", subcore_axis_name="s")
    f = functools.partial(
        pl.kernel,
        out_type=jax.ShapeDtypeStruct((B, C), jnp.float32),
        mesh=mesh,
        scratch_types=[
            pltpu.VMEM((NCHUNKS, CHUNK), jnp.int32),
            pltpu.VMEM((2, CHUNK, C), jnp.float32),
            pltpu.VMEM((2, CHUNK, C), jnp.float32),
            pltpu.VMEM((1, C), jnp.float32),
            pltpu.SemaphoreType.DMA,
            pltpu.SemaphoreType.DMA,
            pltpu.SemaphoreType.DMA,
            pltpu.SemaphoreType.DMA,
            pltpu.SemaphoreType.DMA,
            pltpu.SemaphoreType.DMA,
        ],
    )(_sc_body)
    idx2d = route_index.astype(jnp.int32).reshape(NUM_WORKERS * NCHUNKS, CHUNK)
    return f(input, idx2d, weight, weight_fact)


def kernel(input, route_index, weight, weight_fact):
    return _run(input, route_index, weight, weight_fact)
